# baseline (speedup 1.0000x reference)
import jax
import jax.numpy as jnp
from jax.experimental import pallas as pl


def _copy_body(x_ref, o_ref):
    o_ref[...] = x_ref[...]


def kernel(hidden_state):
    B, S, D = hidden_state.shape
    x = hidden_state.reshape(B * S, D)
    rows = B * S
    block_rows = 1024
    grid = (rows // block_rows,)
    out = pl.pallas_call(
        _copy_body,
        grid=grid,
        in_specs=[pl.BlockSpec((block_rows, D), lambda i: (i, 0))],
        out_specs=pl.BlockSpec((block_rows, D), lambda i: (i, 0)),
        out_shape=jax.ShapeDtypeStruct((rows, D), hidden_state.dtype),
    )(x)
    return out.reshape(B, S, D)
